# baseline (speedup 1.0000x reference)
import functools

import jax
import jax.numpy as jnp
from jax import lax
from jax.experimental import pallas as pl
from jax.experimental.pallas import tpu as pltpu
from jax.experimental.pallas import tpu_sc as plsc

N_NODES = 10000
N_EDGES = 320000
D_FEAT = 128
D_EDGE = 16
HIDDEN = 256
OUT_NF = 128

NW = 32
CH = 80
GCHUNKS = N_EDGES // (NW * CH)
ACHUNKS = N_NODES // CH
NSTRIP = 5
SGROUPS = ((0, 1, 2), (3, 4))
ESTRIP = N_EDGES // NSTRIP
GSCH2 = ESTRIP // (16 * CH)
SCHUNKS = ESTRIP // (16 * CH)


def _silu(x):
    return x * jax.nn.sigmoid(x)


def _pack_halves(x32):
    m = x32.shape[1] // 2
    xr = x32.astype(jnp.bfloat16).astype(jnp.float32)
    return pltpu.pack_elementwise([xr[:, :m], xr[:, m:]],
                                  packed_dtype=jnp.bfloat16)


def _prep_body(h_ref, wa_ref, wb_ref, b1_ref, ha_ref, hb_ref):
    hblk = h_ref[...]
    ha_ref[...] = _pack_halves(
        jnp.dot(hblk, wa_ref[...], preferred_element_type=jnp.float32)
        + b1_ref[...])
    hb_ref[...] = _pack_halves(
        jnp.dot(hblk, wb_ref[...], preferred_element_type=jnp.float32))


def _prep(h, w1a, w1b, b1r):
    blk = 1000
    grid = N_NODES // blk
    return pl.pallas_call(
        _prep_body,
        grid=(grid,),
        in_specs=[
            pl.BlockSpec((blk, D_FEAT), lambda i: (i, 0)),
            pl.BlockSpec((D_FEAT, HIDDEN), lambda i: (0, 0)),
            pl.BlockSpec((D_FEAT, HIDDEN), lambda i: (0, 0)),
            pl.BlockSpec((1, HIDDEN), lambda i: (0, 0)),
        ],
        out_specs=[
            pl.BlockSpec((blk, HIDDEN // 2), lambda i: (i, 0)),
            pl.BlockSpec((blk, HIDDEN // 2), lambda i: (i, 0)),
        ],
        out_shape=[
            jax.ShapeDtypeStruct((N_NODES, HIDDEN // 2), jnp.int32),
            jax.ShapeDtypeStruct((N_NODES, HIDDEN // 2), jnp.int32),
        ],
    )(h, w1a, w1b, b1r)


def _gather_strip(hAB, rc4d, s):
    mesh = plsc.VectorSubcoreMesh(core_axis_name="c", subcore_axis_name="s")

    @functools.partial(
        pl.kernel,
        out_type=jax.ShapeDtypeStruct((2, ESTRIP, HIDDEN // 2), jnp.int32),
        mesh=mesh,
        scratch_types=[
            pltpu.VMEM_SHARED((N_NODES, HIDDEN // 2), jnp.int32),
            pltpu.VMEM((GSCH2, CH), jnp.int32),
            [pltpu.VMEM((CH, HIDDEN // 2), jnp.int32) for _ in range(2)],
            [pltpu.SemaphoreType.DMA for _ in range(2)],
            [pltpu.SemaphoreType.DMA for _ in range(2)],
        ],
    )
    def k(hAB_hbm, rc_hbm, out_hbm, tbl, idx, bA, sG, sO):
        cid = lax.axis_index("c")
        sid = lax.axis_index("s")
        for kk in range((ACHUNKS + 15) // 16):
            ch = sid + kk * 16

            @pl.when(ch < ACHUNKS)
            def _():
                rs = pl.ds(ch * CH, CH)
                pltpu.sync_copy(hAB_hbm.at[cid].at[rs], tbl.at[rs])
        pltpu.sync_copy(rc_hbm.at[cid * (NSTRIP * 16) + s * 16 + sid], idx)
        plsc.subcore_barrier()
        ebase = sid * GSCH2 * CH

        def g_start(j, t):
            pltpu.async_copy(tbl.at[idx.at[j]], bA[t], sG[t])

        def g_wait(j, t):
            pltpu.make_async_copy(tbl.at[idx.at[j]], bA[t], sG[t]).wait()

        def o_dst(j):
            return out_hbm.at[cid].at[pl.ds(ebase + j * CH, CH)]

        def w_wait(j, t):
            pltpu.make_async_copy(bA[t], o_dst(j), sO[t]).wait()

        g_start(0, 0)

        def pair(j2, _):
            for t in range(2):
                j = j2 * 2 + t
                g_wait(j, t)
                pltpu.async_copy(bA[t], o_dst(j), sO[t])

                @pl.when(j > 0)
                def _():
                    w_wait(j - 1, 1 - t)

                @pl.when(j + 1 < GSCH2)
                def _():
                    g_start(j + 1, 1 - t)
            return 0

        lax.fori_loop(0, GSCH2 // 2, pair, 0)
        w_wait(GSCH2 - 1, 1)

    return k(hAB, rc4d)


def _unpack2(gw):
    lo = pltpu.unpack_elementwise(gw, index=0, packed_dtype=jnp.bfloat16,
                                  unpacked_dtype=jnp.float32)
    hi = pltpu.unpack_elementwise(gw, index=1, packed_dtype=jnp.bfloat16,
                                  unpacked_dtype=jnp.float32)
    return lo, hi


def _edge_compute(g1_ref, g2_ref, attr_ref, w1c_ref, w2_ref, b2_ref):
    half = HIDDEN // 2
    a0, a1 = _unpack2(jnp.squeeze(g1_ref[...], axis=0))
    b0, b1 = _unpack2(jnp.squeeze(g2_ref[...], axis=0))
    attrc = jnp.dot(attr_ref[...].astype(jnp.bfloat16), w1c_ref[...],
                    preferred_element_type=jnp.float32)
    t0 = _silu(a0 + b0 + attrc[:, :half]).astype(jnp.bfloat16)
    t1 = _silu(a1 + b1 + attrc[:, half:]).astype(jnp.bfloat16)
    w2 = w2_ref[...]
    m = (jnp.dot(t0, w2[:half], preferred_element_type=jnp.float32)
         + jnp.dot(t1, w2[half:], preferred_element_type=jnp.float32)
         + b2_ref[...])
    return _silu(m)


def _edge_body(g1_ref, g2_ref, attr_ref, w1c_ref, w2_ref, b2_ref,
               strip_ref, full_ref):
    v = _edge_compute(g1_ref, g2_ref, attr_ref, w1c_ref, w2_ref, b2_ref)
    strip_ref[...] = v
    full_ref[...] = v


def _edge_body_aliased(prev_ref, g1_ref, g2_ref, attr_ref, w1c_ref, w2_ref,
                       b2_ref, strip_ref, full_ref):
    del prev_ref
    v = _edge_compute(g1_ref, g2_ref, attr_ref, w1c_ref, w2_ref, b2_ref)
    strip_ref[...] = v
    full_ref[...] = v


def _edge_strip(mij_prev, G1s, G2s, edge_attr, w1c, w2, b2r, s):
    blk = 800
    grid = ESTRIP // blk
    base = s * (ESTRIP // blk)
    body = _edge_body if mij_prev is None else _edge_body_aliased
    in_specs = [
        pl.BlockSpec((1, blk, HIDDEN // 2), lambda i: (0, i, 0)),
        pl.BlockSpec((1, blk, HIDDEN // 2), lambda i: (1, i, 0)),
        pl.BlockSpec((blk, D_EDGE), lambda i: (base + i, 0)),
        pl.BlockSpec((D_EDGE, HIDDEN), lambda i: (0, 0)),
        pl.BlockSpec((HIDDEN, HIDDEN), lambda i: (0, 0)),
        pl.BlockSpec((1, HIDDEN), lambda i: (0, 0)),
    ]
    args = (G1s, G2s, edge_attr, w1c, w2, b2r)
    aliases = {}
    if mij_prev is not None:
        in_specs = [pl.BlockSpec(memory_space=pltpu.MemorySpace.HBM)] + in_specs
        args = (mij_prev,) + args
        aliases = {0: 1}
    return pl.pallas_call(
        body,
        grid=(grid,),
        in_specs=in_specs,
        out_specs=[
            pl.BlockSpec((blk, HIDDEN), lambda i: (i, 0)),
            pl.BlockSpec((blk, HIDDEN), lambda i: (base + i, 0)),
        ],
        out_shape=[
            jax.ShapeDtypeStruct((ESTRIP, HIDDEN), jnp.float32),
            jax.ShapeDtypeStruct((N_EDGES, HIDDEN), jnp.float32),
        ],
        input_output_aliases=aliases,
    )(*args)


def _scatter_group(mijs, row3d16, strips):
    mesh = plsc.VectorSubcoreMesh(core_axis_name="c", subcore_axis_name="s")

    @functools.partial(
        pl.kernel,
        out_type=jax.ShapeDtypeStruct((N_NODES, HIDDEN), jnp.float32),
        mesh=mesh,
        scratch_types=[
            pltpu.VMEM_SHARED((N_NODES, HIDDEN // 2), jnp.float32),
            pltpu.VMEM((SCHUNKS, CH), jnp.int32),
            [pltpu.VMEM((CH, HIDDEN // 2), jnp.float32) for _ in range(2)],
            [pltpu.SemaphoreType.DMA for _ in range(2)],
        ],
    )
    def k(*refs):
        n = len(strips)
        mij_hbms = refs[:n]
        row_hbm, agg_hbm, acc, idx, buf, sem = refs[n:]
        cid = lax.axis_index("c")
        sid = lax.axis_index("s")
        half = HIDDEN // 2

        def zrow(r, _):
            for q in range(half // 16):
                buf[0][r, pl.ds(q * 16, 16)] = jnp.zeros((16,), jnp.float32)
            return 0

        lax.fori_loop(0, CH, zrow, 0)
        for kk in range((ACHUNKS + 15) // 16):
            ch = sid + kk * 16

            @pl.when(ch < ACHUNKS)
            def _():
                pltpu.sync_copy(buf[0], acc.at[pl.ds(ch * CH, CH)])
        plsc.subcore_barrier()

        ebase = sid * SCHUNKS * CH
        for si, s in enumerate(strips):
            mij_hbm = mij_hbms[si]
            pltpu.sync_copy(row_hbm.at[s * 16 + sid], idx)

            def m_src(j):
                return mij_hbm.at[pl.ds(ebase + j * CH, CH),
                                  pl.ds(cid * half, half)]

            pltpu.async_copy(m_src(0), buf[0], sem[0])

            def chunk2(j2, _):
                for t in range(2):
                    j = j2 * 2 + t
                    pltpu.make_async_copy(m_src(j), buf[t], sem[t]).wait()

                    @pl.when(j + 1 < SCHUNKS)
                    def _():
                        pltpu.async_copy(m_src(j + 1), buf[1 - t],
                                         sem[1 - t])
                    pltpu.sync_copy(buf[t], acc.at[idx.at[j]], add=True)
                return 0

            lax.fori_loop(0, SCHUNKS // 2, chunk2, 0)
        plsc.subcore_barrier()

        for kk in range((ACHUNKS + 15) // 16):
            ch = sid + kk * 16

            @pl.when(ch < ACHUNKS)
            def _():
                rs = pl.ds(ch * CH, CH)
                pltpu.sync_copy(acc.at[rs],
                                agg_hbm.at[rs, pl.ds(cid * half, half)])

    return k(*mijs, row3d16)


def _node_body(h_ref, *rest):
    agg_refs = rest[:len(SGROUPS)]
    w3a_ref, w3b_ref, b3_ref, w4_ref, b4_ref, out_ref = rest[len(SGROUPS):]
    hblk = h_ref[...]
    agg = agg_refs[0][...]
    for a in agg_refs[1:]:
        agg = agg + a[...]
    hid = _silu(jnp.dot(hblk, w3a_ref[...], preferred_element_type=jnp.float32)
                + jnp.dot(agg, w3b_ref[...],
                          preferred_element_type=jnp.float32)
                + b3_ref[...])
    out_ref[...] = hblk + jnp.dot(hid, w4_ref[...],
                                  preferred_element_type=jnp.float32) + b4_ref[...]


def _node(h, aggs, w3a, w3b, b3r, w4, b4r):
    blk = 1000
    grid = N_NODES // blk
    return pl.pallas_call(
        _node_body,
        grid=(grid,),
        in_specs=[
            pl.BlockSpec((blk, D_FEAT), lambda i: (i, 0)),
        ] + [
            pl.BlockSpec((blk, HIDDEN), lambda i: (i, 0))
            for _ in range(len(SGROUPS))
        ] + [
            pl.BlockSpec((D_FEAT, HIDDEN), lambda i: (0, 0)),
            pl.BlockSpec((HIDDEN, HIDDEN), lambda i: (0, 0)),
            pl.BlockSpec((1, HIDDEN), lambda i: (0, 0)),
            pl.BlockSpec((HIDDEN, OUT_NF), lambda i: (0, 0)),
            pl.BlockSpec((1, OUT_NF), lambda i: (0, 0)),
        ],
        out_specs=pl.BlockSpec((blk, OUT_NF), lambda i: (i, 0)),
        out_shape=jax.ShapeDtypeStruct((N_NODES, OUT_NF), jnp.float32),
    )(h, *aggs, w3a, w3b, b3r, w4, b4r)


def kernel(h, edge_index, edge_attr, W1, b1, W2, b2, W3, b3, W4, b4):
    ei32 = edge_index.astype(jnp.int32)
    row = ei32[0]
    rc4d = ei32.reshape(2 * NSTRIP * 16, GSCH2, CH)
    row3d16 = row.reshape(NSTRIP * 16, SCHUNKS, CH)

    w1a = W1[:D_FEAT]
    w1b = W1[D_FEAT:2 * D_FEAT]
    w1c = W1[2 * D_FEAT:]
    w3a = W3[:D_FEAT]
    w3b = W3[D_FEAT:]

    hA, hB = _prep(h, w1a, w1b, b1.reshape(1, HIDDEN))

    w1cb = w1c.astype(jnp.bfloat16)
    w2b = W2.astype(jnp.bfloat16)
    b2r = b2.reshape(1, HIDDEN)
    hAB = jnp.stack([hA, hB])
    gs = [_gather_strip(hAB, rc4d, s) for s in range(NSTRIP)]
    mij = None
    mij_strips = []
    for s in range(NSTRIP):
        mij_s, mij = _edge_strip(mij, gs[s], gs[s], edge_attr,
                                 w1cb, w2b, b2r, s)
        mij_strips.append(mij_s)
    aggs = [_scatter_group([mij_strips[s] for s in grp], row3d16, grp)
            for grp in SGROUPS]
    h_new = _node(h, aggs, w3a, w3b, b3.reshape(1, HIDDEN), W4,
                  b4.reshape(1, OUT_NF))
    return (h_new, mij)
